# baseline (speedup 1.0000x reference)
import functools

import jax
import jax.numpy as jnp
from jax import lax
from jax.experimental import pallas as pl
from jax.experimental.pallas import tpu as pltpu
from jax.experimental.pallas import tpu_sc as plsc

MIN_STEPS = -12
MAX_STEPS = 12
LOWER_BIN = MAX_STEPS

B, C, H = 64, 512, 360
OUT = H - MAX_STEPS + MIN_STEPS
NGRP = OUT // 16
R = 64
NB = 2


def _pitch_shift_sc(spectrograms, first_bin):
    mesh = plsc.VectorSubcoreMesh(
        core_axis_name="c", subcore_axis_name="s", num_cores=2)

    @functools.partial(
        pl.kernel,
        out_type=(
            jax.ShapeDtypeStruct((B, C, OUT), jnp.float32),
            jax.ShapeDtypeStruct((B, C, OUT), jnp.float32),
        ),
        mesh=mesh,
        scratch_types=[
            pltpu.VMEM((R, H), jnp.float32),
            pltpu.VMEM((R, OUT), jnp.float32),
            pltpu.VMEM((R, OUT), jnp.float32),
            pltpu.VMEM((16,), jnp.int32),
        ],
        compiler_params=pltpu.CompilerParams(
            needs_layout_passes=False, use_tc_tiling_on_sc=True),
    )
    def k(spec_hbm, fb_hbm, x_hbm, xt_hbm, buf, outx, outxt, fbv):
        cid = lax.axis_index("c")
        sid = lax.axis_index("s")
        wid = sid * 2 + cid
        lane = lax.iota(jnp.int32, 16)
        for bi in range(NB):
            b = wid * NB + bi
            blk = pl.multiple_of((b // 16) * 16, 16)
            pltpu.sync_copy(fb_hbm.at[pl.ds(blk, 16)], fbv)
            off = jnp.sum(jnp.where(lane == (b % 16), fbv[...], 0))
            colt = [off + g * 16 + lane for g in range(NGRP)]
            colx = [LOWER_BIN + g * 16 + lane for g in range(NGRP)]
            for r0 in range(0, C, R):
                pltpu.sync_copy(spec_hbm.at[b, pl.ds(r0, R), :], buf)

                def row_body(r, _):
                    rvec = jnp.full((16,), r, dtype=jnp.int32)
                    for g in range(NGRP):
                        c0 = g * 16
                        vx = plsc.load_gather(buf, [rvec, colx[g]])
                        vt = plsc.load_gather(buf, [rvec, colt[g]])
                        outx[r, pl.ds(c0, 16)] = vx
                        outxt[r, pl.ds(c0, 16)] = vt
                    return _

                lax.fori_loop(0, R, row_body, None)
                pltpu.sync_copy(outx, x_hbm.at[b, pl.ds(r0, R), :])
                pltpu.sync_copy(outxt, xt_hbm.at[b, pl.ds(r0, R), :])

    return k(spectrograms, first_bin)


def kernel(spectrograms):
    batch_size = spectrograms.shape[0]
    k = jax.random.fold_in(jax.random.key(0), 1)
    n_steps = jax.random.randint(k, (batch_size,), MIN_STEPS, MAX_STEPS + 1,
                                 dtype=jnp.int32)
    first_bin = (LOWER_BIN - n_steps).astype(jnp.int32)
    x, xt = _pitch_shift_sc(spectrograms, first_bin)
    return (x, xt, n_steps)


# speedup vs baseline: 3.5624x; 3.5624x over previous
import functools

import jax
import jax.numpy as jnp
import numpy as np
from jax import lax
from jax.experimental import pallas as pl
from jax.experimental.pallas import tpu as pltpu
from jax.experimental.pallas import tpu_sc as plsc

MIN_STEPS = -12
MAX_STEPS = 12
LOWER_BIN = MAX_STEPS

B, C, H = 64, 512, 360
OUT = H - MAX_STEPS + MIN_STEPS
RJ = 112
NSLOT = 2
NB = 2


def _pitch_shift_sc(sp_t, first_bin):
    mesh = plsc.VectorSubcoreMesh(
        core_axis_name="c", subcore_axis_name="s", num_cores=2)

    @functools.partial(
        pl.kernel,
        out_type=(
            jax.ShapeDtypeStruct((B, OUT, C), jnp.float32),
            jax.ShapeDtypeStruct((B, OUT, C), jnp.float32),
        ),
        mesh=mesh,
        scratch_types=(
            [pltpu.VMEM((RJ, C), jnp.float32)] * NSLOT
            + [pltpu.VMEM((RJ,), jnp.int32)] * NSLOT
            + [pltpu.VMEM((16,), jnp.int32)]
            + [pltpu.SemaphoreType.DMA] * (2 * NSLOT)
        ),
        compiler_params=pltpu.CompilerParams(
            needs_layout_passes=False, use_tc_tiling_on_sc=True),
    )
    def k(sp_hbm, fb_hbm, x_hbm, xt_hbm, *scratch):
        obufs = scratch[:NSLOT]
        idxs = scratch[NSLOT:2 * NSLOT]
        fbv = scratch[2 * NSLOT]
        sgs = scratch[2 * NSLOT + 1:3 * NSLOT + 1]
        sos = scratch[3 * NSLOT + 1:]
        cid = lax.axis_index("c")
        sid = lax.axis_index("s")
        wid = sid * 2 + cid
        lane = lax.iota(jnp.int32, 16)

        offs = []
        for bi in range(NB):
            b = wid * NB + bi
            blk = pl.multiple_of((b // 16) * 16, 16)
            pltpu.sync_copy(fb_hbm.at[pl.ds(blk, 16)], fbv)
            offs.append(jnp.sum(jnp.where(lane == (b % 16), fbv[...], 0)))

        units = [
            (bi, base_sel, dst, j0)
            for bi in range(NB)
            for base_sel, dst in ((0, x_hbm), (1, xt_hbm))
            for j0 in range(0, OUT, RJ)
        ]

        gathers = [None] * len(units)
        out_pending = [None] * NSLOT

        def launch(v):
            bi, base_sel, dst, j0 = units[v]
            slot = v % NSLOT
            if out_pending[slot] is not None:
                out_pending[slot].wait()
                out_pending[slot] = None
            base = LOWER_BIN if base_sel == 0 else offs[bi]
            jin0 = base + j0
            for g in range(RJ // 16):
                idxs[slot][pl.ds(g * 16, 16)] = jin0 + g * 16 + lane
            cp = pltpu.make_async_copy(
                sp_hbm.at[wid * NB + bi].at[idxs[slot]],
                obufs[slot], sgs[slot])
            cp.start()
            gathers[v] = cp

        def drain(v):
            bi, base_sel, dst, j0 = units[v]
            slot = v % NSLOT
            gathers[v].wait()
            cp = pltpu.make_async_copy(
                obufs[slot],
                dst.at[wid * NB + bi, pl.ds(j0, RJ), :],
                sos[slot])
            cp.start()
            out_pending[slot] = cp

        launch(0)
        for v in range(1, len(units)):
            launch(v)
            drain(v - 1)
        drain(len(units) - 1)
        for cp in out_pending:
            if cp is not None:
                cp.wait()

    return k(sp_t, first_bin)


def kernel(spectrograms):
    batch_size = spectrograms.shape[0]
    k = jax.random.fold_in(jax.random.key(0), 1)
    n_steps = jax.random.randint(k, (batch_size,), MIN_STEPS, MAX_STEPS + 1,
                                 dtype=jnp.int32)
    first_bin = (LOWER_BIN - n_steps).astype(jnp.int32)
    sp_t = jnp.swapaxes(spectrograms, 1, 2)
    x_t, xt_t = _pitch_shift_sc(sp_t, first_bin)
    return (jnp.swapaxes(x_t, 1, 2), jnp.swapaxes(xt_t, 1, 2), n_steps)
